# baseline (speedup 1.0000x reference)
import jax
import jax.numpy as jnp
from jax import lax
from jax.experimental import pallas as pl
from jax.experimental.pallas import tpu as pltpu

_C = 8
_W = 512
_T = 16
_NCH = 32


def _sig_kernel(p_ref, o1, o2, o3, o4, dc_ref):
    B = p_ref.shape[1]
    f32 = jnp.float32

    lane = lax.broadcasted_iota(jnp.int32, (_C, _W), 1)
    row = lax.broadcasted_iota(jnp.int32, (_C, _W), 0)
    lane8 = lax.broadcasted_iota(jnp.int32, (_C, 64), 1)
    row8 = lax.broadcasted_iota(jnp.int32, (_C, 64), 0)
    e23 = jnp.concatenate(
        [((lane >> 6) == row).astype(f32),
         ((lane8 >> 3) == row8).astype(f32)], axis=1)

    o3[...] = jnp.zeros((B, _W), f32)
    o4[...] = jnp.zeros((B, _C * _W), f32)

    def outer(c, carry):
        xs = p_ref[pl.ds(c * _T, _T)]
        xe = p_ref[pl.ds(c * _T + 1, _T)]
        dxc = xe - xs
        dc = jnp.dot(dxc.reshape(_T * B, _C), e23,
                     preferred_element_type=f32)
        dc_ref[...] = dc.reshape(_T, B, _W + 64)

        def inner(tl, sc):
            s1, s2 = sc
            t = c * _T + tl
            dx = p_ref[t + 1] - p_ref[t]
            pats = dc_ref[tl]
            d3 = pats[:, :_W]
            r8 = pats[:, _W:]

            s3v = o3[...]
            ct = s1 + 0.25 * dx
            gt = s2 + (1.0 / 3.0) * (r8 * pltpu.repeat(ct, 8, axis=1))
            h = s3v + d3 * pltpu.repeat(0.5 * gt, 8, axis=1)
            for j in range(_C):
                o4[:, _W * j : _W * (j + 1)] += dx[:, j : j + 1] * h
            cv = s1 + (1.0 / 3.0) * dx
            dv = s2 + 0.5 * (r8 * pltpu.repeat(cv, 8, axis=1))
            o3[...] = s3v + d3 * pltpu.repeat(dv, 8, axis=1)
            av = s1 + 0.5 * dx
            s2n = s2 + r8 * pltpu.repeat(av, 8, axis=1)
            s1n = s1 + dx
            return (s1n, s2n)

        return lax.fori_loop(0, _T, inner, carry)

    init = (jnp.zeros((B, _C), f32), jnp.zeros((B, 64), f32))
    s1, s2 = lax.fori_loop(0, _NCH, outer, init)
    o1[...] = s1
    o2[...] = s2


def kernel(path):
    n, length, c = path.shape
    pt = jnp.swapaxes(path, 0, 1)
    pt = jnp.concatenate([pt, pt[-1:]], axis=0)
    grid_n = 2
    B = n // grid_n
    out_shape = (
        jax.ShapeDtypeStruct((n, _C), jnp.float32),
        jax.ShapeDtypeStruct((n, 64), jnp.float32),
        jax.ShapeDtypeStruct((n, _W), jnp.float32),
        jax.ShapeDtypeStruct((n, _C * _W), jnp.float32),
    )
    s1, s2r, s3r, s4r = pl.pallas_call(
        _sig_kernel,
        grid=(grid_n,),
        in_specs=[pl.BlockSpec((length + 1, B, c), lambda i: (0, i, 0))],
        out_specs=(
            pl.BlockSpec((B, _C), lambda i: (i, 0)),
            pl.BlockSpec((B, 64), lambda i: (i, 0)),
            pl.BlockSpec((B, _W), lambda i: (i, 0)),
            pl.BlockSpec((B, _C * _W), lambda i: (i, 0)),
        ),
        out_shape=out_shape,
        scratch_shapes=[pltpu.VMEM((_T, B, _W + 64), jnp.float32)],
        compiler_params=pltpu.CompilerParams(
            dimension_semantics=("parallel",),
        ),
        name="signature_scan",
    )(pt)
    s2 = s2r.reshape(n, 8, 8).transpose(0, 2, 1).reshape(n, 64)
    s3 = s3r.reshape(n, 8, 8, 8).transpose(0, 3, 2, 1).reshape(n, 512)
    s4 = s4r.reshape(n, 8, 8, 8, 8).transpose(0, 4, 3, 2, 1).reshape(n, 4096)
    return jnp.concatenate([s1, s2, s3, s4], axis=-1)


# speedup vs baseline: 2.0450x; 2.0450x over previous
import jax
import jax.numpy as jnp
from jax import lax
from jax.experimental import pallas as pl
from jax.experimental.pallas import tpu as pltpu

_C = 8
_W = 512


def _sig_kernel(p_ref, o1, o2, o3, o4):
    B = p_ref.shape[1]
    f32 = jnp.float32

    lane = lax.broadcasted_iota(jnp.int32, (_C, 3 * _W), 1)
    row = lax.broadcasted_iota(jnp.int32, (_C, 3 * _W), 0)
    e123 = (((lane & 7) == row) & (lane < _W)).astype(f32) \
        + ((((lane >> 3) & 7) == row) & (_W <= lane) & (lane < 2 * _W)).astype(f32) \
        + ((((lane >> 6) & 7) == row) & (2 * _W <= lane)).astype(f32)

    o3[...] = jnp.zeros((B, _W), f32)
    o4[...] = jnp.zeros((B, _C * _W), f32)

    def pats(dx):
        d = jnp.dot(dx, e123, preferred_element_type=f32)
        return d[:, :_W], d[:, _W : 2 * _W], d[:, 2 * _W :]

    def halfstep(dx, d1, d2, d3, s1, s2):
        s3v = o3[...]
        ct = s1 + 0.25 * d1
        gt = s2 + (1.0 / 3.0) * (d2 * ct)
        h = s3v + 0.5 * (d3 * gt)
        for j in range(_C):
            o4[:, _W * j : _W * (j + 1)] += dx[:, j : j + 1] * h
        cv = s1 + (1.0 / 3.0) * d1
        dv = s2 + 0.5 * (d2 * cv)
        o3[...] = s3v + d3 * dv
        av = s1 + 0.5 * d1
        s2n = s2 + d2 * av
        s1n = s1 + d1
        return s1n, s2n

    x0 = p_ref[0]
    x1 = p_ref[1]
    x2 = p_ref[2]
    dxa0 = x1 - x0
    dxb0 = x2 - x1
    da0 = pats(dxa0)
    db0 = pats(dxb0)
    init = (
        dxa0, da0[0], da0[1], da0[2],
        dxb0, db0[0], db0[1], db0[2],
        x2,
        jnp.zeros((B, _W), f32),
        jnp.zeros((B, _W), f32),
    )
    steps = p_ref.shape[0] - 1
    nbody = steps // 2

    def step(i, carry):
        (dxa, d1a, d2a, d3a, dxb, d1b, d2b, d3b, x, s1, s2) = carry
        t = 2 * i
        xn1 = p_ref[jnp.minimum(t + 3, steps)]
        xn2 = p_ref[jnp.minimum(t + 4, steps)]
        dxan = xn1 - x
        dxbn = xn2 - xn1
        d1an, d2an, d3an = pats(dxan)
        d1bn, d2bn, d3bn = pats(dxbn)

        s1, s2 = halfstep(dxa, d1a, d2a, d3a, s1, s2)
        s1, s2 = halfstep(dxb, d1b, d2b, d3b, s1, s2)
        return (dxan, d1an, d2an, d3an, dxbn, d1bn, d2bn, d3bn,
                xn2, s1, s2)

    carry = lax.fori_loop(0, nbody, step, init)
    s1, s2 = carry[9], carry[10]
    o1[...] = s1
    o2[...] = s2


def kernel(path):
    n, length, c = path.shape
    pt = jnp.swapaxes(path, 0, 1)
    pt = jnp.concatenate([pt, pt[-1:]], axis=0)
    grid_n = 2
    B = n // grid_n
    out_shape = (
        jax.ShapeDtypeStruct((n, _W), jnp.float32),
        jax.ShapeDtypeStruct((n, _W), jnp.float32),
        jax.ShapeDtypeStruct((n, _W), jnp.float32),
        jax.ShapeDtypeStruct((n, _C * _W), jnp.float32),
    )
    s1t, s2t, s3r, s4r = pl.pallas_call(
        _sig_kernel,
        grid=(grid_n,),
        in_specs=[pl.BlockSpec((length + 1, B, c), lambda i: (0, i, 0))],
        out_specs=(
            pl.BlockSpec((B, _W), lambda i: (i, 0)),
            pl.BlockSpec((B, _W), lambda i: (i, 0)),
            pl.BlockSpec((B, _W), lambda i: (i, 0)),
            pl.BlockSpec((B, _C * _W), lambda i: (i, 0)),
        ),
        out_shape=out_shape,
        compiler_params=pltpu.CompilerParams(
            dimension_semantics=("parallel",),
        ),
        name="signature_scan",
    )(pt)
    s1 = s1t[:, :8]
    s2 = s2t[:, :64].reshape(n, 8, 8).transpose(0, 2, 1).reshape(n, 64)
    s3 = s3r.reshape(n, 8, 8, 8).transpose(0, 3, 2, 1).reshape(n, 512)
    s4 = s4r.reshape(n, 8, 8, 8, 8).transpose(0, 4, 3, 2, 1).reshape(n, 4096)
    return jnp.concatenate([s1, s2, s3, s4], axis=-1)
